# baseline (speedup 1.0000x reference)
import functools

import jax
import jax.numpy as jnp
from jax import lax
from jax.experimental import pallas as pl
from jax.experimental.pallas import tpu as pltpu
from jax.experimental.pallas import tpu_sc as plsc

_V = 50000
_E = 1600000
_S = 400000

_NC, _NS, _L = 2, 16, 16
_NW = _NC * _NS

_VPAD = 50048
_VROWS = 391
_EROWS_W = 400
_EROWS = _NW * _EROWS_W
_EPAD = _EROWS * 128
_RB = 8
_NBLK = _EROWS_W // _RB
_SPAD = 409600
_SSLICE = _SPAD // _NS
_ZB = 1600
_MROWS = 400
_SROWS = _SPAD // 128


def _prep_body(x_ref, v_ref):
    x = x_ref[...]
    r = lax.broadcasted_iota(jnp.int32, (_VROWS, 128), 0)
    c = lax.broadcasted_iota(jnp.int32, (_VROWS, 128), 1)
    valid = (r * 128 + c) < _V
    absx = jnp.abs(x)
    loga = jnp.where(absx > 0, jnp.log(absx), jnp.float32(-1e30))
    t = 2 + (x < 0).astype(jnp.int32)
    packed = (lax.bitcast_convert_type(loga, jnp.int32) & ~jnp.int32(3)) | t
    v_ref[...] = jnp.where(valid, packed, 0)


_prep = pl.pallas_call(
    _prep_body,
    out_shape=jax.ShapeDtypeStruct((_VROWS, 128), jnp.int32),
)


def _merge_body(a_ref, t_ref, o_ref):
    a = a_ref[0] + a_ref[1]
    t = t_ref[0] + t_ref[1]
    sign = (1 - 2 * (t & 1)).astype(jnp.float32)
    o_ref[...] = jnp.where(t > 0, sign * jnp.exp(a), jnp.float32(0.0))


_merge = pl.pallas_call(
    _merge_body,
    grid=(_SROWS // _MROWS,),
    in_specs=[
        pl.BlockSpec((2, _MROWS, 128), lambda i: (0, i, 0)),
        pl.BlockSpec((2, _MROWS, 128), lambda i: (0, i, 0)),
    ],
    out_specs=pl.BlockSpec((_MROWS, 128), lambda i: (i, 0)),
    out_shape=jax.ShapeDtypeStruct((_SROWS, 128), jnp.float32),
)


_mesh = plsc.VectorSubcoreMesh(
    core_axis_name="c", subcore_axis_name="s", num_cores=_NC, num_subcores=_NS
)


@functools.partial(
    pl.kernel,
    out_type=(
        jax.ShapeDtypeStruct((_NC, _SPAD), jnp.float32),
        jax.ShapeDtypeStruct((_NC, _SPAD), jnp.int32),
    ),
    mesh=_mesh,
    compiler_params=pltpu.CompilerParams(needs_layout_passes=False),
    scratch_types=[
        pltpu.VMEM((_VPAD,), jnp.int32),
        pltpu.VMEM((2, _RB, 128), jnp.int32),
        pltpu.VMEM((2, _RB, 128), jnp.int32),
        pltpu.VMEM((2, _RB, 128), jnp.int32),
        pltpu.VMEM((2, _RB, 128), jnp.float32),
        pltpu.VMEM((2, _RB, 128), jnp.int32),
        pltpu.VMEM((_ZB,), jnp.float32),
        pltpu.VMEM((_ZB,), jnp.int32),
        pltpu.VMEM_SHARED((_SPAD,), jnp.float32),
        pltpu.VMEM_SHARED((_SPAD,), jnp.int32),
        pltpu.SemaphoreType.DMA,
        pltpu.SemaphoreType.DMA,
        pltpu.SemaphoreType.DMA,
        pltpu.SemaphoreType.DMA,
    ],
)
def _sc_seg(v_hbm, ptrs_hbm, csr_hbm, outa_hbm, outt_hbm,
            v_tab, pbuf, cbuf, sbuf, abuf, tbuf, zf, zi, sha, sht,
            sin0, sin1, ssc0, ssc1):
    cid = lax.axis_index("c")
    sid = lax.axis_index("s")
    wid = sid * _NC + cid
    sin = (sin0, sin1)
    ssc = (ssc0, ssc1)

    pltpu.sync_copy(v_hbm, v_tab)

    zerof = jnp.zeros((_L,), jnp.float32)
    zeroi = jnp.zeros((_L,), jnp.int32)
    for j in range(_ZB // _L):
        zf[pl.ds(j * _L, _L)] = zerof
        zi[pl.ds(j * _L, _L)] = zeroi
    base = sid * _SSLICE
    for j in range(_SSLICE // _ZB):
        pltpu.sync_copy(zf, sha.at[pl.ds(base + j * _ZB, _ZB)])
        pltpu.sync_copy(zi, sht.at[pl.ds(base + j * _ZB, _ZB)])
    plsc.subcore_barrier()

    row0 = wid * _EROWS_W

    def start_in(b, slot):
        r0 = row0 + b * _RB
        pltpu.async_copy(ptrs_hbm.at[pl.ds(r0, _RB)], pbuf.at[slot], sin[slot])
        pltpu.async_copy(csr_hbm.at[pl.ds(r0, _RB)], cbuf.at[slot], sin[slot])

    def wait_in(b, slot):
        r0 = row0 + b * _RB
        pltpu.make_async_copy(
            ptrs_hbm.at[pl.ds(r0, _RB)], pbuf.at[slot], sin[slot]).wait()
        pltpu.make_async_copy(
            csr_hbm.at[pl.ds(r0, _RB)], cbuf.at[slot], sin[slot]).wait()

    def drain_sc(slot):
        for r in range(_RB):
            pltpu.make_async_copy(
                abuf.at[slot, r], sha.at[sbuf.at[slot, r]], ssc[slot]).wait()
            pltpu.make_async_copy(
                tbuf.at[slot, r], sht.at[sbuf.at[slot, r]], ssc[slot]).wait()

    start_in(0, 0)
    start_in(1, 1)

    _NQ = 128 // _L
    perm = [lax.iota(jnp.int32, _L) * _NQ + q for q in range(_NQ)]

    def blk2(i2, carry):
        for slot in (0, 1):
            b = i2 * 2 + slot
            wait_in(b, slot)

            @pl.when(i2 > 0)
            def _():
                drain_sc(slot)

            for r in range(_RB):
                arow = abuf.at[slot, r]
                trow = tbuf.at[slot, r]
                srow = sbuf.at[slot, r]
                for q in range(_NQ):
                    idx = pbuf[slot, r, pl.ds(q * _L, _L)]
                    v = plsc.load_gather(v_tab, [idx])
                    plsc.store_scatter(
                        arow, [perm[q]],
                        lax.bitcast_convert_type(v & ~jnp.int32(3),
                                                 jnp.float32))
                    plsc.store_scatter(trow, [perm[q]], v & jnp.int32(3))
                    plsc.store_scatter(
                        srow, [perm[q]], cbuf[slot, r, pl.ds(q * _L, _L)])
            for r in range(_RB):
                pltpu.async_copy(
                    abuf.at[slot, r], sha.at[sbuf.at[slot, r]], ssc[slot],
                    add=True)
                pltpu.async_copy(
                    tbuf.at[slot, r], sht.at[sbuf.at[slot, r]], ssc[slot],
                    add=True)

            @pl.when(i2 < _NBLK // 2 - 1)
            def _():
                start_in(b + 2, slot)
        return carry

    lax.fori_loop(0, _NBLK // 2, blk2, 0)
    drain_sc(0)
    drain_sc(1)
    plsc.subcore_barrier()

    pltpu.sync_copy(sha.at[pl.ds(base, _SSLICE)],
                    outa_hbm.at[cid, pl.ds(base, _SSLICE)])
    pltpu.sync_copy(sht.at[pl.ds(base, _SSLICE)],
                    outt_hbm.at[cid, pl.ds(base, _SSLICE)])


def kernel(x, ptrs, csr):
    x_pad = jnp.zeros((_VPAD,), jnp.float32).at[:_V].set(x)
    v_tab = _prep(x_pad.reshape(_VROWS, 128))
    pad = _EPAD - _E
    ptrs_p = jnp.concatenate(
        [ptrs, jnp.full((pad,), _V, jnp.int32)]).reshape(_EROWS, 128)
    csr_p = jnp.concatenate(
        [csr, jnp.zeros((pad,), jnp.int32)]).reshape(_EROWS, 128)
    outa, outt = _sc_seg(v_tab.reshape(_VPAD), ptrs_p, csr_p)
    out = _merge(outa.reshape(_NC, _SROWS, 128),
                 outt.reshape(_NC, _SROWS, 128))
    return out.reshape(_SPAD)[:_S]


# speedup vs baseline: 1.0675x; 1.0675x over previous
import functools

import jax
import jax.numpy as jnp
from jax import lax
from jax.experimental import pallas as pl
from jax.experimental.pallas import tpu as pltpu
from jax.experimental.pallas import tpu_sc as plsc

_V = 50000
_E = 1600000
_S = 400000

_NC, _NS, _L = 2, 16, 16
_NW = _NC * _NS

_VPAD = 50048
_VROWS = 391
_EROWS_W = 400
_EROWS = _NW * _EROWS_W
_EPAD = _EROWS * 128
_RB = 16
_NBLK = _EROWS_W // _RB
_SPAD = 409600
_SSLICE = _SPAD // _NS
_ZB = 1600
_MROWS = 400
_SROWS = _SPAD // 128


def _prep_body(x_ref, v_ref):
    x = x_ref[...]
    r = lax.broadcasted_iota(jnp.int32, (_VROWS, 128), 0)
    c = lax.broadcasted_iota(jnp.int32, (_VROWS, 128), 1)
    valid = (r * 128 + c) < _V
    absx = jnp.abs(x)
    loga = jnp.where(absx > 0, jnp.log(absx), jnp.float32(-1e30))
    t = 2 + (x < 0).astype(jnp.int32)
    packed = (lax.bitcast_convert_type(loga, jnp.int32) & ~jnp.int32(3)) | t
    v_ref[...] = jnp.where(valid, packed, 0)


_prep = pl.pallas_call(
    _prep_body,
    out_shape=jax.ShapeDtypeStruct((_VROWS, 128), jnp.int32),
)


def _merge_body(a_ref, t_ref, o_ref):
    a = a_ref[0] + a_ref[1]
    t = t_ref[0] + t_ref[1]
    sign = (1 - 2 * (t & 1)).astype(jnp.float32)
    o_ref[...] = jnp.where(t > 0, sign * jnp.exp(a), jnp.float32(0.0))


_merge = pl.pallas_call(
    _merge_body,
    grid=(_SROWS // _MROWS,),
    in_specs=[
        pl.BlockSpec((2, _MROWS, 128), lambda i: (0, i, 0)),
        pl.BlockSpec((2, _MROWS, 128), lambda i: (0, i, 0)),
    ],
    out_specs=pl.BlockSpec((_MROWS, 128), lambda i: (i, 0)),
    out_shape=jax.ShapeDtypeStruct((_SROWS, 128), jnp.float32),
)


_mesh = plsc.VectorSubcoreMesh(
    core_axis_name="c", subcore_axis_name="s", num_cores=_NC, num_subcores=_NS
)


@functools.partial(
    pl.kernel,
    out_type=(
        jax.ShapeDtypeStruct((_NC, _SPAD), jnp.float32),
        jax.ShapeDtypeStruct((_NC, _SPAD), jnp.int32),
    ),
    mesh=_mesh,
    compiler_params=pltpu.CompilerParams(needs_layout_passes=False),
    scratch_types=[
        pltpu.VMEM((_VPAD,), jnp.int32),
        pltpu.VMEM((2, _RB, 128), jnp.int32),
        pltpu.VMEM((2, _RB, 128), jnp.int32),
        pltpu.VMEM((2, _RB, 128), jnp.int32),
        pltpu.VMEM((2, _RB, 128), jnp.float32),
        pltpu.VMEM((2, _RB, 128), jnp.int32),
        pltpu.VMEM((_ZB,), jnp.float32),
        pltpu.VMEM((_ZB,), jnp.int32),
        pltpu.VMEM_SHARED((_SPAD,), jnp.float32),
        pltpu.VMEM_SHARED((_SPAD,), jnp.int32),
        pltpu.SemaphoreType.DMA,
        pltpu.SemaphoreType.DMA,
        pltpu.SemaphoreType.DMA,
        pltpu.SemaphoreType.DMA,
    ],
)
def _sc_seg(v_hbm, ptrs_hbm, csr_hbm, outa_hbm, outt_hbm,
            v_tab, pbuf, cbuf, sbuf, abuf, tbuf, zf, zi, sha, sht,
            sin0, sin1, ssc0, ssc1):
    cid = lax.axis_index("c")
    sid = lax.axis_index("s")
    wid = sid * _NC + cid
    sin = (sin0, sin1)
    ssc = (ssc0, ssc1)

    pltpu.sync_copy(v_hbm, v_tab)

    zerof = jnp.zeros((_L,), jnp.float32)
    zeroi = jnp.zeros((_L,), jnp.int32)
    for j in range(_ZB // _L):
        zf[pl.ds(j * _L, _L)] = zerof
        zi[pl.ds(j * _L, _L)] = zeroi
    base = sid * _SSLICE
    for j in range(_SSLICE // _ZB):
        pltpu.sync_copy(zf, sha.at[pl.ds(base + j * _ZB, _ZB)])
        pltpu.sync_copy(zi, sht.at[pl.ds(base + j * _ZB, _ZB)])
    plsc.subcore_barrier()

    row0 = wid * _EROWS_W

    def start_in(b, slot):
        r0 = row0 + b * _RB
        pltpu.async_copy(ptrs_hbm.at[pl.ds(r0, _RB)], pbuf.at[slot], sin[slot])
        pltpu.async_copy(csr_hbm.at[pl.ds(r0, _RB)], cbuf.at[slot], sin[slot])

    def wait_in(b, slot):
        r0 = row0 + b * _RB
        pltpu.make_async_copy(
            ptrs_hbm.at[pl.ds(r0, _RB)], pbuf.at[slot], sin[slot]).wait()
        pltpu.make_async_copy(
            csr_hbm.at[pl.ds(r0, _RB)], cbuf.at[slot], sin[slot]).wait()

    def drain_sc(slot):
        for r in range(_RB):
            pltpu.make_async_copy(
                abuf.at[slot, r], sha.at[sbuf.at[slot, r]], ssc[slot]).wait()
            pltpu.make_async_copy(
                tbuf.at[slot, r], sht.at[sbuf.at[slot, r]], ssc[slot]).wait()

    start_in(0, 0)
    start_in(1, 1)

    def blk2(i2, carry):
        for slot in (0, 1):
            b = i2 * 2 + slot
            wait_in(b, slot)

            @pl.when(i2 > 0)
            def _():
                drain_sc(slot)

            for r in range(_RB):
                for q in range(128 // _L):
                    idx = pbuf[slot, r, pl.ds(q * _L, _L)]
                    v = plsc.load_gather(v_tab, [idx])
                    abuf[slot, r, pl.ds(q * _L, _L)] = lax.bitcast_convert_type(
                        v & ~jnp.int32(3), jnp.float32)
                    tbuf[slot, r, pl.ds(q * _L, _L)] = v & jnp.int32(3)
                    sbuf[slot, r, pl.ds(q * _L, _L)] = (
                        cbuf[slot, r, pl.ds(q * _L, _L)])
            for r in range(_RB):
                pltpu.async_copy(
                    abuf.at[slot, r], sha.at[sbuf.at[slot, r]], ssc[slot],
                    add=True)
                pltpu.async_copy(
                    tbuf.at[slot, r], sht.at[sbuf.at[slot, r]], ssc[slot],
                    add=True)

            @pl.when(i2 < _NBLK // 2 - 1)
            def _():
                start_in(b + 2, slot)
        return carry

    lax.fori_loop(0, _NBLK // 2, blk2, 0)
    drain_sc(0)
    drain_sc(1)
    plsc.subcore_barrier()

    pltpu.sync_copy(sha.at[pl.ds(base, _SSLICE)],
                    outa_hbm.at[cid, pl.ds(base, _SSLICE)])
    pltpu.sync_copy(sht.at[pl.ds(base, _SSLICE)],
                    outt_hbm.at[cid, pl.ds(base, _SSLICE)])


def kernel(x, ptrs, csr):
    x_pad = jnp.zeros((_VPAD,), jnp.float32).at[:_V].set(x)
    v_tab = _prep(x_pad.reshape(_VROWS, 128))
    pad = _EPAD - _E
    ptrs_p = jnp.concatenate(
        [ptrs, jnp.full((pad,), _V, jnp.int32)]).reshape(_EROWS, 128)
    csr_p = jnp.concatenate(
        [csr, jnp.zeros((pad,), jnp.int32)]).reshape(_EROWS, 128)
    outa, outt = _sc_seg(v_tab.reshape(_VPAD), ptrs_p, csr_p)
    out = _merge(outa.reshape(_NC, _SROWS, 128),
                 outt.reshape(_NC, _SROWS, 128))
    return out.reshape(_SPAD)[:_S]


# speedup vs baseline: 1.1276x; 1.0563x over previous
import functools

import jax
import jax.numpy as jnp
from jax import lax
from jax.experimental import pallas as pl
from jax.experimental.pallas import tpu as pltpu
from jax.experimental.pallas import tpu_sc as plsc

_V = 50000
_E = 1600000
_S = 400000

_NC, _NS, _L = 2, 16, 16
_NW = _NC * _NS

_VPAD = 50048
_VROWS = 391
_EROWS_W = 400
_EROWS = _E // 128
_TROW0 = (_NW - 1) * _EROWS_W
_RB = 8
_NBLK = _EROWS_W // _RB
_SPAD = 409600
_SSLICE = _SPAD // _NS
_ZB = 1600
_MROWS = 400
_SROWS = _SPAD // 128


def _prep_body(x_ref, v_ref):
    x = x_ref[...]
    r = lax.broadcasted_iota(jnp.int32, (_VROWS, 128), 0)
    c = lax.broadcasted_iota(jnp.int32, (_VROWS, 128), 1)
    valid = (r * 128 + c) < _V
    absx = jnp.abs(x)
    loga = jnp.where(absx > 0, jnp.log(absx), jnp.float32(-1e30))
    t = 2 + (x < 0).astype(jnp.int32)
    packed = (lax.bitcast_convert_type(loga, jnp.int32) & ~jnp.int32(3)) | t
    v_ref[...] = jnp.where(valid, packed, 0)


_prep = pl.pallas_call(
    _prep_body,
    out_shape=jax.ShapeDtypeStruct((_VROWS, 128), jnp.int32),
)


def _merge_body(a_ref, t_ref, o_ref):
    a = a_ref[0] + a_ref[1]
    t = t_ref[0] + t_ref[1]
    sign = (1 - 2 * (t & 1)).astype(jnp.float32)
    o_ref[...] = jnp.where(t > 0, sign * jnp.exp(a), jnp.float32(0.0))


_MBLK = _SPAD // 8

_merge = pl.pallas_call(
    _merge_body,
    grid=(_SPAD // _MBLK,),
    in_specs=[
        pl.BlockSpec((2, _MBLK), lambda i: (0, i)),
        pl.BlockSpec((2, _MBLK), lambda i: (0, i)),
    ],
    out_specs=pl.BlockSpec((_MBLK,), lambda i: (i,)),
    out_shape=jax.ShapeDtypeStruct((_SPAD,), jnp.float32),
)


_mesh = plsc.VectorSubcoreMesh(
    core_axis_name="c", subcore_axis_name="s", num_cores=_NC, num_subcores=_NS
)


@functools.partial(
    pl.kernel,
    out_type=(
        jax.ShapeDtypeStruct((_NC, _SPAD), jnp.float32),
        jax.ShapeDtypeStruct((_NC, _SPAD), jnp.int32),
    ),
    mesh=_mesh,
    compiler_params=pltpu.CompilerParams(needs_layout_passes=False),
    scratch_types=[
        pltpu.VMEM((_VPAD,), jnp.int32),
        pltpu.VMEM((2, _RB, 128), jnp.int32),
        pltpu.VMEM((2, _RB, 128), jnp.int32),
        pltpu.VMEM((2, _RB, 128), jnp.int32),
        pltpu.VMEM((2, _RB, 128), jnp.float32),
        pltpu.VMEM((2, _RB, 128), jnp.int32),
        pltpu.VMEM((_ZB,), jnp.float32),
        pltpu.VMEM((_ZB,), jnp.int32),
        pltpu.VMEM_SHARED((_SPAD,), jnp.float32),
        pltpu.VMEM_SHARED((_SPAD,), jnp.int32),
        pltpu.SemaphoreType.DMA,
        pltpu.SemaphoreType.DMA,
        pltpu.SemaphoreType.DMA,
        pltpu.SemaphoreType.DMA,
    ],
)
def _sc_seg(v_hbm, ptrs_hbm, csr_hbm, ptail_hbm, ctail_hbm,
            outa_hbm, outt_hbm,
            v_tab, pbuf, cbuf, sbuf, abuf, tbuf, zf, zi, sha, sht,
            sin0, sin1, ssc0, ssc1):
    cid = lax.axis_index("c")
    sid = lax.axis_index("s")
    wid = sid * _NC + cid
    sin = (sin0, sin1)
    ssc = (ssc0, ssc1)

    pltpu.sync_copy(v_hbm, v_tab)

    zerof = jnp.zeros((_L,), jnp.float32)
    zeroi = jnp.zeros((_L,), jnp.int32)
    for j in range(_ZB // _L):
        zf[pl.ds(j * _L, _L)] = zerof
        zi[pl.ds(j * _L, _L)] = zeroi
    base = sid * _SSLICE
    for j in range(_SSLICE // _ZB):
        pltpu.sync_copy(zf, sha.at[pl.ds(base + j * _ZB, _ZB)])
        pltpu.sync_copy(zi, sht.at[pl.ds(base + j * _ZB, _ZB)])
    plsc.subcore_barrier()

    row0 = wid * _EROWS_W
    is_tail = wid == _NW - 1

    def start_in(b, slot):
        @pl.when(is_tail)
        def _():
            r0 = b * _RB
            pltpu.async_copy(
                ptail_hbm.at[pl.ds(r0, _RB)], pbuf.at[slot], sin[slot])
            pltpu.async_copy(
                ctail_hbm.at[pl.ds(r0, _RB)], cbuf.at[slot], sin[slot])

        @pl.when(jnp.logical_not(is_tail))
        def _():
            r0 = row0 + b * _RB
            pltpu.async_copy(
                ptrs_hbm.at[pl.ds(r0, _RB)], pbuf.at[slot], sin[slot])
            pltpu.async_copy(
                csr_hbm.at[pl.ds(r0, _RB)], cbuf.at[slot], sin[slot])

    def wait_in(b, slot):
        del b
        pltpu.make_async_copy(
            ptrs_hbm.at[pl.ds(0, _RB)], pbuf.at[slot], sin[slot]).wait()
        pltpu.make_async_copy(
            csr_hbm.at[pl.ds(0, _RB)], cbuf.at[slot], sin[slot]).wait()

    def drain_sc(slot):
        for r in range(_RB):
            pltpu.make_async_copy(
                abuf.at[slot, r], sha.at[sbuf.at[slot, r]], ssc[slot]).wait()
            pltpu.make_async_copy(
                tbuf.at[slot, r], sht.at[sbuf.at[slot, r]], ssc[slot]).wait()

    start_in(0, 0)
    start_in(1, 1)

    def blk2(i2, carry):
        for slot in (0, 1):
            b = i2 * 2 + slot
            wait_in(b, slot)

            @pl.when(i2 > 0)
            def _():
                drain_sc(slot)

            for r in range(_RB):
                for q in range(128 // _L):
                    idx = pbuf[slot, r, pl.ds(q * _L, _L)]
                    v = plsc.load_gather(v_tab, [idx])
                    abuf[slot, r, pl.ds(q * _L, _L)] = lax.bitcast_convert_type(
                        v & ~jnp.int32(3), jnp.float32)
                    tbuf[slot, r, pl.ds(q * _L, _L)] = v & jnp.int32(3)
                    sbuf[slot, r, pl.ds(q * _L, _L)] = (
                        cbuf[slot, r, pl.ds(q * _L, _L)])
            for r in range(_RB):
                pltpu.async_copy(
                    abuf.at[slot, r], sha.at[sbuf.at[slot, r]], ssc[slot],
                    add=True)
                pltpu.async_copy(
                    tbuf.at[slot, r], sht.at[sbuf.at[slot, r]], ssc[slot],
                    add=True)

            @pl.when(i2 < _NBLK // 2 - 1)
            def _():
                start_in(b + 2, slot)
        return carry

    lax.fori_loop(0, _NBLK // 2, blk2, 0)
    drain_sc(0)
    drain_sc(1)
    plsc.subcore_barrier()

    pltpu.sync_copy(sha.at[pl.ds(base, _SSLICE)],
                    outa_hbm.at[cid, pl.ds(base, _SSLICE)])
    pltpu.sync_copy(sht.at[pl.ds(base, _SSLICE)],
                    outt_hbm.at[cid, pl.ds(base, _SSLICE)])


def kernel(x, ptrs, csr):
    x_pad = jnp.zeros((_VPAD,), jnp.float32).at[:_V].set(x)
    v_tab = _prep(x_pad.reshape(_VROWS, 128))
    tail_e = _EROWS_W * 128
    real_e = _E - _TROW0 * 128
    ptail = jnp.full((tail_e,), _V, jnp.int32).at[:real_e].set(
        ptrs[_TROW0 * 128:]).reshape(_EROWS_W, 128)
    ctail = jnp.zeros((tail_e,), jnp.int32).at[:real_e].set(
        csr[_TROW0 * 128:]).reshape(_EROWS_W, 128)
    outa, outt = _sc_seg(v_tab.reshape(_VPAD),
                         ptrs.reshape(_EROWS, 128),
                         csr.reshape(_EROWS, 128), ptail, ctail)
    return _merge(outa, outt)[:_S]


# speedup vs baseline: 1.1303x; 1.0024x over previous
import functools

import jax
import jax.numpy as jnp
from jax import lax
from jax.experimental import pallas as pl
from jax.experimental.pallas import tpu as pltpu
from jax.experimental.pallas import tpu_sc as plsc

_V = 50000
_E = 1600000
_S = 400000

_NC, _NS, _L = 2, 16, 16
_NW = _NC * _NS

_VPAD = 50048
_VROWS = 391
_EROWS_W = 400
_EROWS = _E // 128
_TROW0 = (_NW - 1) * _EROWS_W
_RB = 8
_NBLK = _EROWS_W // _RB
_SPAD = 409600
_SSLICE = _SPAD // _NS
_ZB = 1600
_MROWS = 400
_SROWS = _SPAD // 128


def _prep_body(x_ref, v_ref):
    x = x_ref[...]
    r = lax.broadcasted_iota(jnp.int32, (_VROWS, 128), 0)
    c = lax.broadcasted_iota(jnp.int32, (_VROWS, 128), 1)
    valid = (r * 128 + c) < _V
    absx = jnp.abs(x)
    loga = jnp.where(absx > 0, jnp.log(absx), jnp.float32(-1e30))
    t = 2 + (x < 0).astype(jnp.int32)
    packed = (lax.bitcast_convert_type(loga, jnp.int32) & ~jnp.int32(3)) | t
    v_ref[...] = jnp.where(valid, packed, 0)


_prep = pl.pallas_call(
    _prep_body,
    out_shape=jax.ShapeDtypeStruct((_VROWS, 128), jnp.int32),
)


def _merge_body(a_ref, t_ref, o_ref):
    a = a_ref[0] + a_ref[1]
    t = t_ref[0] + t_ref[1]
    sign = (1 - 2 * (t & 1)).astype(jnp.float32)
    o_ref[...] = jnp.where(t > 0, sign * jnp.exp(a), jnp.float32(0.0))


_MBLK = _SPAD // 8

_merge = pl.pallas_call(
    _merge_body,
    grid=(_SPAD // _MBLK,),
    in_specs=[
        pl.BlockSpec((2, _MBLK), lambda i: (0, i)),
        pl.BlockSpec((2, _MBLK), lambda i: (0, i)),
    ],
    out_specs=pl.BlockSpec((_MBLK,), lambda i: (i,)),
    out_shape=jax.ShapeDtypeStruct((_SPAD,), jnp.float32),
)


_mesh = plsc.VectorSubcoreMesh(
    core_axis_name="c", subcore_axis_name="s", num_cores=_NC, num_subcores=_NS
)


@functools.partial(
    pl.kernel,
    out_type=(
        jax.ShapeDtypeStruct((_NC, _SPAD), jnp.float32),
        jax.ShapeDtypeStruct((_NC, _SPAD), jnp.int32),
    ),
    mesh=_mesh,
    compiler_params=pltpu.CompilerParams(needs_layout_passes=False),
    scratch_types=[
        pltpu.VMEM((_VPAD,), jnp.int32),
        pltpu.VMEM((2, _RB, 128), jnp.int32),
        pltpu.VMEM((2, _RB, 128), jnp.int32),
        pltpu.VMEM((2, _RB, 128), jnp.int32),
        pltpu.VMEM((2, _RB, 128), jnp.float32),
        pltpu.VMEM((2, _RB, 128), jnp.int32),
        pltpu.VMEM((_ZB,), jnp.float32),
        pltpu.VMEM((_ZB,), jnp.int32),
        pltpu.VMEM_SHARED((_SPAD,), jnp.float32),
        pltpu.VMEM_SHARED((_SPAD,), jnp.int32),
        pltpu.SemaphoreType.DMA,
        pltpu.SemaphoreType.DMA,
        pltpu.SemaphoreType.DMA,
        pltpu.SemaphoreType.DMA,
    ],
)
def _sc_seg(v_hbm, ptrs_hbm, csr_hbm, ptail_hbm, ctail_hbm,
            outa_hbm, outt_hbm,
            v_tab, pbuf, cbuf, sbuf, abuf, tbuf, zf, zi, sha, sht,
            sin0, sin1, ssc0, ssc1):
    cid = lax.axis_index("c")
    sid = lax.axis_index("s")
    wid = sid * _NC + cid
    sin = (sin0, sin1)
    ssc = (ssc0, ssc1)

    pltpu.sync_copy(v_hbm, v_tab)

    zerof = jnp.zeros((_L,), jnp.float32)
    zeroi = jnp.zeros((_L,), jnp.int32)
    for j in range(_ZB // _L):
        zf[pl.ds(j * _L, _L)] = zerof
        zi[pl.ds(j * _L, _L)] = zeroi
    base = sid * _SSLICE
    for j in range(_SSLICE // _ZB):
        pltpu.sync_copy(zf, sha.at[pl.ds(base + j * _ZB, _ZB)])
        pltpu.sync_copy(zi, sht.at[pl.ds(base + j * _ZB, _ZB)])
    plsc.subcore_barrier()

    row0 = wid * _EROWS_W
    is_tail = wid == _NW - 1

    def start_in(b, slot):
        @pl.when(is_tail)
        def _():
            r0 = b * _RB
            pltpu.async_copy(
                ptail_hbm.at[pl.ds(r0, _RB)], pbuf.at[slot], sin[slot])
            pltpu.async_copy(
                ctail_hbm.at[pl.ds(r0, _RB)], cbuf.at[slot], sin[slot])

        @pl.when(jnp.logical_not(is_tail))
        def _():
            r0 = row0 + b * _RB
            pltpu.async_copy(
                ptrs_hbm.at[pl.ds(r0, _RB)], pbuf.at[slot], sin[slot])
            pltpu.async_copy(
                csr_hbm.at[pl.ds(r0, _RB)], cbuf.at[slot], sin[slot])

    def wait_in(b, slot):
        @pl.when(is_tail)
        def _():
            r0 = b * _RB
            pltpu.make_async_copy(
                ptail_hbm.at[pl.ds(r0, _RB)], pbuf.at[slot], sin[slot]).wait()
            pltpu.make_async_copy(
                ctail_hbm.at[pl.ds(r0, _RB)], cbuf.at[slot], sin[slot]).wait()

        @pl.when(jnp.logical_not(is_tail))
        def _():
            r0 = row0 + b * _RB
            pltpu.make_async_copy(
                ptrs_hbm.at[pl.ds(r0, _RB)], pbuf.at[slot], sin[slot]).wait()
            pltpu.make_async_copy(
                csr_hbm.at[pl.ds(r0, _RB)], cbuf.at[slot], sin[slot]).wait()

    def drain_sc(slot):
        for r in range(_RB):
            pltpu.make_async_copy(
                abuf.at[slot, r], sha.at[sbuf.at[slot, r]], ssc[slot]).wait()
            pltpu.make_async_copy(
                tbuf.at[slot, r], sht.at[sbuf.at[slot, r]], ssc[slot]).wait()

    start_in(0, 0)
    start_in(1, 1)

    def blk2(i2, carry):
        for slot in (0, 1):
            b = i2 * 2 + slot
            wait_in(b, slot)

            @pl.when(i2 > 0)
            def _():
                drain_sc(slot)

            for r in range(_RB):
                for q in range(128 // _L):
                    idx = pbuf[slot, r, pl.ds(q * _L, _L)]
                    v = plsc.load_gather(v_tab, [idx])
                    abuf[slot, r, pl.ds(q * _L, _L)] = lax.bitcast_convert_type(
                        v & ~jnp.int32(3), jnp.float32)
                    tbuf[slot, r, pl.ds(q * _L, _L)] = v & jnp.int32(3)
                    sbuf[slot, r, pl.ds(q * _L, _L)] = (
                        cbuf[slot, r, pl.ds(q * _L, _L)])
            for r in range(_RB):
                pltpu.async_copy(
                    abuf.at[slot, r], sha.at[sbuf.at[slot, r]], ssc[slot],
                    add=True)
                pltpu.async_copy(
                    tbuf.at[slot, r], sht.at[sbuf.at[slot, r]], ssc[slot],
                    add=True)

            @pl.when(i2 < _NBLK // 2 - 1)
            def _():
                start_in(b + 2, slot)
        return carry

    lax.fori_loop(0, _NBLK // 2, blk2, 0)
    drain_sc(0)
    drain_sc(1)
    plsc.subcore_barrier()

    pltpu.sync_copy(sha.at[pl.ds(base, _SSLICE)],
                    outa_hbm.at[cid, pl.ds(base, _SSLICE)])
    pltpu.sync_copy(sht.at[pl.ds(base, _SSLICE)],
                    outt_hbm.at[cid, pl.ds(base, _SSLICE)])


def kernel(x, ptrs, csr):
    x_pad = jnp.zeros((_VPAD,), jnp.float32).at[:_V].set(x)
    v_tab = _prep(x_pad.reshape(_VROWS, 128))
    tail_e = _EROWS_W * 128
    real_e = _E - _TROW0 * 128
    ptail = jnp.full((tail_e,), _V, jnp.int32).at[:real_e].set(
        ptrs[_TROW0 * 128:]).reshape(_EROWS_W, 128)
    ctail = jnp.zeros((tail_e,), jnp.int32).at[:real_e].set(
        csr[_TROW0 * 128:]).reshape(_EROWS_W, 128)
    outa, outt = _sc_seg(v_tab.reshape(_VPAD),
                         ptrs.reshape(_EROWS, 128),
                         csr.reshape(_EROWS, 128), ptail, ctail)
    return _merge(outa, outt)[:_S]
